# baseline (speedup 1.0000x reference)
import jax
import jax.numpy as jnp
from jax.experimental import pallas as pl
from jax.experimental.pallas import tpu as pltpu

_NB = 72


def _add_kernel(x_ref, e_ref, o_ref):
    o_ref[...] = x_ref[...] + e_ref[...][:, None, None, :]


def kernel(x, emb_weight):
    B, N, T, D = x.shape
    xt = jnp.transpose(x, (1, 2, 0, 3))
    out = pl.pallas_call(
        _add_kernel,
        grid=(pl.cdiv(N, _NB),),
        in_specs=[
            pl.BlockSpec((_NB, T, B, D), lambda j: (j, 0, 0, 0)),
            pl.BlockSpec((_NB, D), lambda j: (j, 0)),
        ],
        out_specs=pl.BlockSpec((_NB, T, B, D), lambda j: (j, 0, 0, 0)),
        out_shape=jax.ShapeDtypeStruct((N, T, B, D), x.dtype),
        compiler_params=pltpu.CompilerParams(
            dimension_semantics=("parallel",),
        ),
    )(xt, emb_weight)
    return jnp.transpose(out, (2, 0, 1, 3))


# speedup vs baseline: 1.0040x; 1.0040x over previous
import jax
import jax.numpy as jnp
from jax.experimental import pallas as pl
from jax.experimental.pallas import tpu as pltpu

_NB = 48


def _add_kernel(x_ref, e_ref, o_ref):
    o_ref[...] = x_ref[...] + e_ref[...][:, None, None, :]


def kernel(x, emb_weight):
    B, N, T, D = x.shape
    xt = jnp.transpose(x, (1, 2, 0, 3))
    out = pl.pallas_call(
        _add_kernel,
        grid=(pl.cdiv(N, _NB),),
        in_specs=[
            pl.BlockSpec((_NB, T, B, D), lambda j: (j, 0, 0, 0)),
            pl.BlockSpec((_NB, D), lambda j: (j, 0)),
        ],
        out_specs=pl.BlockSpec((_NB, T, B, D), lambda j: (j, 0, 0, 0)),
        out_shape=jax.ShapeDtypeStruct((N, T, B, D), x.dtype),
        compiler_params=pltpu.CompilerParams(
            dimension_semantics=("parallel",),
        ),
    )(xt, emb_weight)
    return jnp.transpose(out, (2, 0, 1, 3))


# speedup vs baseline: 1.0044x; 1.0004x over previous
import jax
import jax.numpy as jnp
from jax.experimental import pallas as pl
from jax.experimental.pallas import tpu as pltpu

_NB = 64


def _add_kernel(x_ref, e_ref, o_ref):
    o_ref[...] = x_ref[...] + e_ref[...][:, None, None, :]


def kernel(x, emb_weight):
    B, N, T, D = x.shape
    xt = jnp.transpose(x, (1, 2, 0, 3))
    out = pl.pallas_call(
        _add_kernel,
        grid=(pl.cdiv(N, _NB),),
        in_specs=[
            pl.BlockSpec((_NB, T, B, D), lambda j: (j, 0, 0, 0)),
            pl.BlockSpec((_NB, D), lambda j: (j, 0)),
        ],
        out_specs=pl.BlockSpec((_NB, T, B, D), lambda j: (j, 0, 0, 0)),
        out_shape=jax.ShapeDtypeStruct((N, T, B, D), x.dtype),
        compiler_params=pltpu.CompilerParams(
            dimension_semantics=("parallel",),
        ),
    )(xt, emb_weight)
    return jnp.transpose(out, (2, 0, 1, 3))
